# baseline (speedup 1.0000x reference)
import functools

import jax
import jax.numpy as jnp
import numpy as np
from jax.experimental import pallas as pl
from jax.experimental.pallas import tpu as pltpu

_B = 32768
_N = 12
_C = 64
_NC = _N * _C
_NN = _N * _N
_THRED = 0.1

_ADJ = np.array([
    [0, 0, 0, 1, 0, 1, 1, 1, 1, 1, 1, 1],
    [0, 0, 0, 1, 0, 1, 1, 1, 1, 1, 1, 1],
    [0, 0, 0, 1, 0, 1, 1, 1, 1, 1, 1, 1],
    [1, 1, 1, 0, 1, 1, 1, 1, 1, 1, 1, 1],
    [0, 0, 0, 1, 0, 1, 1, 1, 1, 1, 1, 1],
    [1, 1, 1, 1, 1, 0, 1, 1, 1, 0, 0, 0],
    [1, 1, 1, 1, 1, 1, 0, 0, 0, 1, 1, 1],
    [1, 1, 1, 1, 1, 1, 0, 0, 0, 1, 1, 1],
    [1, 1, 1, 1, 1, 1, 0, 0, 0, 1, 1, 1],
    [1, 1, 1, 1, 1, 0, 1, 1, 1, 0, 0, 0],
    [1, 1, 1, 1, 1, 0, 1, 1, 1, 0, 0, 0],
    [1, 1, 1, 1, 1, 0, 1, 1, 1, 0, 0, 0],
], dtype=np.float32)


def _gat_block(x_ref, m_ref, W2_ref, E_ref, S_ref, G_ref, adj_ref, eye_ref,
               o_ref, *, bb):
    x = x_ref[...].reshape(bb, _NC)
    m = m_ref[...].reshape(bb, _NN)
    W2 = W2_ref[...]
    feat = jnp.concatenate(
        [jnp.dot(x[:, k * 128:(k + 1) * 128], W2,
                 preferred_element_type=jnp.float32) for k in range(_NC // 128)],
        axis=1)
    e = jnp.dot(x, E_ref[...], preferred_element_type=jnp.float32)

    e = jnp.where(e >= 0, e, 0.2 * e)
    adjv = adj_ref[...] * m + eye_ref[...]
    p = jnp.where(adjv > _THRED, jnp.exp(e), 0.0)

    s = jnp.dot(p, S_ref[...], preferred_element_type=jnp.float32)
    G = G_ref[...]
    inv_s = jnp.dot(1.0 / s, G, preferred_element_type=jnp.float32)

    for i in range(_N):
        pi = jnp.dot(p[:, i * _N:(i + 1) * _N], G,
                     preferred_element_type=jnp.float32)
        t = pi * feat
        a = t[:, :256] + t[:, 256:512] + t[:, 512:768]
        a = a[:, :128] + a[:, 128:]
        acc = a[:, :_C] + a[:, _C:]
        o_ref[:, i, :] = (
            acc * inv_s[:, i * _C:(i + 1) * _C] + x[:, i * _C:(i + 1) * _C])


def kernel(x, adj_mask, W, a_l, a_r):
    bb = 512
    grid = (_B // bb,)

    eye12 = jnp.eye(_N, dtype=jnp.float32)
    W2 = jnp.kron(jnp.eye(2, dtype=jnp.float32), W)
    wl = W @ a_l
    wr = W @ a_r
    K1 = jnp.kron(eye12, jnp.ones((1, _N), jnp.float32))
    K2 = jnp.tile(eye12, (1, _N))
    E = jnp.kron(eye12, wl[:, None]) @ K1 + jnp.kron(eye12, wr[:, None]) @ K2
    S = jnp.kron(eye12, jnp.ones((_N, 1), jnp.float32))
    G = jnp.kron(eye12, jnp.ones((1, _C), jnp.float32))

    adj_row = jnp.asarray(_ADJ).reshape(1, _NN)
    eye_row = eye12.reshape(1, _NN)

    fn = pl.pallas_call(
        functools.partial(_gat_block, bb=bb),
        grid=grid,
        in_specs=[
            pl.BlockSpec((bb, _N, _C), lambda b: (b, 0, 0)),
            pl.BlockSpec((bb, _N, _N), lambda b: (b, 0, 0)),
            pl.BlockSpec((128, 128), lambda b: (0, 0)),
            pl.BlockSpec((_NC, _NN), lambda b: (0, 0)),
            pl.BlockSpec((_NN, _N), lambda b: (0, 0)),
            pl.BlockSpec((_N, _NC), lambda b: (0, 0)),
            pl.BlockSpec((1, _NN), lambda b: (0, 0)),
            pl.BlockSpec((1, _NN), lambda b: (0, 0)),
        ],
        out_specs=pl.BlockSpec((bb, _N, _C), lambda b: (b, 0, 0)),
        out_shape=jax.ShapeDtypeStruct((_B, _N, _C), jnp.float32),
        compiler_params=pltpu.CompilerParams(
            dimension_semantics=("parallel",)),
    )
    return fn(x, adj_mask, W2, E, S, G, adj_row, eye_row)


# speedup vs baseline: 2.0153x; 2.0153x over previous
import functools

import jax
import jax.numpy as jnp
import numpy as np
from jax.experimental import pallas as pl

_B = 32768
_N = 12
_C = 64
_NC = _N * _C
_NN = _N * _N
_THRED = 0.1

_ADJ = np.array([
    [0, 0, 0, 1, 0, 1, 1, 1, 1, 1, 1, 1],
    [0, 0, 0, 1, 0, 1, 1, 1, 1, 1, 1, 1],
    [0, 0, 0, 1, 0, 1, 1, 1, 1, 1, 1, 1],
    [1, 1, 1, 0, 1, 1, 1, 1, 1, 1, 1, 1],
    [0, 0, 0, 1, 0, 1, 1, 1, 1, 1, 1, 1],
    [1, 1, 1, 1, 1, 0, 1, 1, 1, 0, 0, 0],
    [1, 1, 1, 1, 1, 1, 0, 0, 0, 1, 1, 1],
    [1, 1, 1, 1, 1, 1, 0, 0, 0, 1, 1, 1],
    [1, 1, 1, 1, 1, 1, 0, 0, 0, 1, 1, 1],
    [1, 1, 1, 1, 1, 0, 1, 1, 1, 0, 0, 0],
    [1, 1, 1, 1, 1, 0, 1, 1, 1, 0, 0, 0],
    [1, 1, 1, 1, 1, 0, 1, 1, 1, 0, 0, 0],
], dtype=np.float32)


def _gat_block(x_ref, m_ref, W2_ref, E_ref, S_ref, G_ref, adj_ref, eye_ref,
               o_ref, *, bb):
    x = x_ref[...]
    W2 = W2_ref[...]
    feat = jnp.concatenate(
        [jnp.dot(x[:, k * 128:(k + 1) * 128], W2,
                 preferred_element_type=jnp.float32) for k in range(_NC // 128)],
        axis=1)
    e = jnp.dot(x, E_ref[...], preferred_element_type=jnp.float32)

    e = jnp.where(e >= 0, e, 0.2 * e)
    adjv = adj_ref[...] * m_ref[...] + eye_ref[...]
    p = jnp.where(adjv > _THRED, jnp.exp(e), 0.0)

    s = jnp.dot(p, S_ref[...], preferred_element_type=jnp.float32)
    G = G_ref[...]
    inv_s = jnp.dot(1.0 / s, G, preferred_element_type=jnp.float32)

    for i in range(_N):
        pi = jnp.dot(p[:, i * _N:(i + 1) * _N], G,
                     preferred_element_type=jnp.float32)
        t = pi * feat
        a = t[:, :256] + t[:, 256:512] + t[:, 512:768]
        a = a[:, :128] + a[:, 128:]
        acc = a[:, :_C] + a[:, _C:]
        o_ref[:, i * _C:(i + 1) * _C] = (
            acc * inv_s[:, i * _C:(i + 1) * _C] + x[:, i * _C:(i + 1) * _C])


def kernel(x, adj_mask, W, a_l, a_r):
    bb = 1024
    grid = (_B // bb,)

    eye12 = jnp.eye(_N, dtype=jnp.float32)
    W2 = jnp.kron(jnp.eye(2, dtype=jnp.float32), W)
    wl = W @ a_l
    wr = W @ a_r
    K1 = jnp.kron(eye12, jnp.ones((1, _N), jnp.float32))
    K2 = jnp.tile(eye12, (1, _N))
    E = jnp.kron(eye12, wl[:, None]) @ K1 + jnp.kron(eye12, wr[:, None]) @ K2
    S = jnp.kron(eye12, jnp.ones((_N, 1), jnp.float32))
    G = jnp.kron(eye12, jnp.ones((1, _C), jnp.float32))

    adj_row = jnp.asarray(_ADJ).reshape(1, _NN)
    eye_row = eye12.reshape(1, _NN)

    fn = pl.pallas_call(
        functools.partial(_gat_block, bb=bb),
        grid=grid,
        in_specs=[
            pl.BlockSpec((bb, _NC), lambda b: (b, 0)),
            pl.BlockSpec((bb, _NN), lambda b: (b, 0)),
            pl.BlockSpec((128, 128), lambda b: (0, 0)),
            pl.BlockSpec((_NC, _NN), lambda b: (0, 0)),
            pl.BlockSpec((_NN, _N), lambda b: (0, 0)),
            pl.BlockSpec((_N, _NC), lambda b: (0, 0)),
            pl.BlockSpec((1, _NN), lambda b: (0, 0)),
            pl.BlockSpec((1, _NN), lambda b: (0, 0)),
        ],
        out_specs=pl.BlockSpec((bb, _NC), lambda b: (b, 0)),
        out_shape=jax.ShapeDtypeStruct((_B, _NC), jnp.float32),
    )
    out = fn(x.reshape(_B, _NC), adj_mask.reshape(_B, _NN),
             W2, E, S, G, adj_row, eye_row)
    return out.reshape(_B, _N, _C)


# speedup vs baseline: 2.0163x; 1.0005x over previous
import functools

import jax
import jax.numpy as jnp
import numpy as np
from jax.experimental import pallas as pl
from jax.experimental.pallas import tpu as pltpu

_B = 32768
_N = 12
_C = 64
_NC = _N * _C
_NN = _N * _N
_THRED = 0.1

_ADJ = np.array([
    [0, 0, 0, 1, 0, 1, 1, 1, 1, 1, 1, 1],
    [0, 0, 0, 1, 0, 1, 1, 1, 1, 1, 1, 1],
    [0, 0, 0, 1, 0, 1, 1, 1, 1, 1, 1, 1],
    [1, 1, 1, 0, 1, 1, 1, 1, 1, 1, 1, 1],
    [0, 0, 0, 1, 0, 1, 1, 1, 1, 1, 1, 1],
    [1, 1, 1, 1, 1, 0, 1, 1, 1, 0, 0, 0],
    [1, 1, 1, 1, 1, 1, 0, 0, 0, 1, 1, 1],
    [1, 1, 1, 1, 1, 1, 0, 0, 0, 1, 1, 1],
    [1, 1, 1, 1, 1, 1, 0, 0, 0, 1, 1, 1],
    [1, 1, 1, 1, 1, 0, 1, 1, 1, 0, 0, 0],
    [1, 1, 1, 1, 1, 0, 1, 1, 1, 0, 0, 0],
    [1, 1, 1, 1, 1, 0, 1, 1, 1, 0, 0, 0],
], dtype=np.float32)


def _gat_block(x_ref, m_ref, W2_ref, E_ref, S_ref, G_ref, adj_ref, eye_ref,
               o_ref, *, bb):
    x = x_ref[...]
    W2 = W2_ref[...]
    feat = jnp.concatenate(
        [jnp.dot(x[:, k * 128:(k + 1) * 128], W2,
                 preferred_element_type=jnp.float32) for k in range(_NC // 128)],
        axis=1)
    e = jnp.dot(x, E_ref[...], preferred_element_type=jnp.float32)

    e = jnp.where(e >= 0, e, 0.2 * e)
    adjv = adj_ref[...] * m_ref[...] + eye_ref[...]
    p = jnp.where(adjv > _THRED, jnp.exp(e), 0.0)

    s = jnp.dot(p, S_ref[...], preferred_element_type=jnp.float32)
    G = G_ref[...]
    inv_s = jnp.dot(1.0 / s, G, preferred_element_type=jnp.float32)

    for i in range(_N):
        pi = jnp.dot(p[:, i * _N:(i + 1) * _N], G,
                     preferred_element_type=jnp.float32)
        t = pi * feat
        a = t[:, :256] + t[:, 256:512] + t[:, 512:768]
        a = a[:, :128] + a[:, 128:]
        acc = a[:, :_C] + a[:, _C:]
        o_ref[:, i * _C:(i + 1) * _C] = (
            acc * inv_s[:, i * _C:(i + 1) * _C] + x[:, i * _C:(i + 1) * _C])


def kernel(x, adj_mask, W, a_l, a_r):
    bb = 1024
    grid = (_B // bb,)

    eye12 = jnp.eye(_N, dtype=jnp.float32)
    W2 = jnp.kron(jnp.eye(2, dtype=jnp.float32), W)
    wl = W @ a_l
    wr = W @ a_r
    K1 = jnp.kron(eye12, jnp.ones((1, _N), jnp.float32))
    K2 = jnp.tile(eye12, (1, _N))
    E = jnp.kron(eye12, wl[:, None]) @ K1 + jnp.kron(eye12, wr[:, None]) @ K2
    S = jnp.kron(eye12, jnp.ones((_N, 1), jnp.float32))
    G = jnp.kron(eye12, jnp.ones((1, _C), jnp.float32))

    adj_row = jnp.asarray(_ADJ).reshape(1, _NN)
    eye_row = eye12.reshape(1, _NN)

    fn = pl.pallas_call(
        functools.partial(_gat_block, bb=bb),
        grid=grid,
        in_specs=[
            pl.BlockSpec((bb, _NC), lambda b: (b, 0)),
            pl.BlockSpec((bb, _NN), lambda b: (b, 0)),
            pl.BlockSpec((128, 128), lambda b: (0, 0)),
            pl.BlockSpec((_NC, _NN), lambda b: (0, 0)),
            pl.BlockSpec((_NN, _N), lambda b: (0, 0)),
            pl.BlockSpec((_N, _NC), lambda b: (0, 0)),
            pl.BlockSpec((1, _NN), lambda b: (0, 0)),
            pl.BlockSpec((1, _NN), lambda b: (0, 0)),
        ],
        out_specs=pl.BlockSpec((bb, _NC), lambda b: (b, 0)),
        out_shape=jax.ShapeDtypeStruct((_B, _NC), jnp.float32),
        compiler_params=pltpu.CompilerParams(
            allow_input_fusion=[True] * 8),
    )
    out = fn(x.reshape(_B, _NC), adj_mask.reshape(_B, _NN),
             W2, E, S, G, adj_row, eye_row)
    return out.reshape(_B, _N, _C)
